# baseline (speedup 1.0000x reference)
import functools

import jax
import jax.numpy as jnp
from jax import lax
from jax.experimental import pallas as pl
from jax.experimental.pallas import tpu as pltpu
from jax.experimental.pallas import tpu_sc as plsc

_NC = 2
_NS = 16
_NW = _NC * _NS

_CH = 128
_G = 2
_CB = _CH * _G
_DP = 128


def kernel(input_ids, wte):
    B, L = input_ids.shape
    V, D = wte.shape
    N = B * L
    idx_flat = input_ids.reshape(N).astype(jnp.int32)
    lines = wte.reshape(V // 2, _DP)

    n_per_w = N // _NW
    n_blk = n_per_w // _CB
    assert n_per_w % _CB == 0 and n_blk % 2 == 0

    line_idx = idx_flat >> 1

    mesh = plsc.VectorSubcoreMesh(core_axis_name="c", subcore_axis_name="s")

    @functools.partial(
        pl.kernel,
        mesh=mesh,
        out_type=jax.ShapeDtypeStruct((N, _DP), jnp.float32),
        scratch_types=[
            pltpu.VMEM((n_per_w,), jnp.int32),
            pltpu.VMEM((_CB, _DP), jnp.float32),
            pltpu.VMEM((_CB, _DP), jnp.float32),
            pltpu.SemaphoreType.DMA,
            pltpu.SemaphoreType.DMA,
            pltpu.SemaphoreType.DMA,
            pltpu.SemaphoreType.DMA,
        ],
    )
    def emb(idx_hbm, table_hbm, out_hbm, idx_v, buf0, buf1, sg0, sg1, sw0, sw1):
        wid = lax.axis_index("s") * _NC + lax.axis_index("c")
        base = wid * n_per_w
        pltpu.sync_copy(idx_hbm.at[pl.ds(base, n_per_w)], idx_v)

        bufs = (buf0, buf1)
        sgs = (sg0, sg1)
        sws = (sw0, sw1)

        def fire_gathers(blk, buf, sem):
            off = blk * _CB
            for j in range(_G):
                pltpu.async_copy(
                    table_hbm.at[idx_v.at[pl.ds(off + j * _CH, _CH)]],
                    buf.at[pl.ds(j * _CH, _CH)],
                    sem,
                )

        def drain_gathers(buf, sem):
            pltpu.make_async_copy(table_hbm.at[pl.ds(0, _CB)], buf, sem).wait()

        def fire_writeout(blk, buf, sem):
            pltpu.async_copy(buf, out_hbm.at[pl.ds(base + blk * _CB, _CB)], sem)

        def drain_writeout(buf, sem):
            pltpu.make_async_copy(buf, out_hbm.at[pl.ds(base, _CB)], sem).wait()

        fire_gathers(0, buf0, sg0)

        def body(t, carry):
            for b in range(2):
                blk = 2 * t + b
                p, q = b % 2, (b + 1) % 2
                drain_gathers(bufs[p], sgs[p])

                @pl.when(jnp.logical_and(blk >= 1, blk + 1 < n_blk))
                def _():
                    drain_writeout(bufs[q], sws[q])

                @pl.when(blk + 1 < n_blk)
                def _():
                    fire_gathers(blk + 1, bufs[q], sgs[q])

                fire_writeout(blk, bufs[p], sws[p])
            return carry

        lax.fori_loop(0, n_blk // 2, body, 0)
        drain_writeout(bufs[0], sws[0])
        drain_writeout(bufs[1], sws[1])

    out = emb(line_idx, lines)
    odd = (idx_flat & 1)[:, None] == 1
    out64 = jnp.where(odd, out[:, D:], out[:, :D])
    return out64.reshape(B, L, D)


# speedup vs baseline: 1.1378x; 1.1378x over previous
import functools

import jax
import jax.numpy as jnp
from jax import lax
from jax.experimental import pallas as pl
from jax.experimental.pallas import tpu as pltpu
from jax.experimental.pallas import tpu_sc as plsc

_NC = 2
_NS = 16
_NW = _NC * _NS

_CH = 128
_G = 2
_CB = _CH * _G
_DP = 128


def _make_lines(wte_t, V):
    KB = 2048
    grid = (V + KB - 1) // KB
    n_lines = grid * (KB // 2)

    def tr(x_ref, o_ref):
        xt = x_ref[...].T
        pieces = [
            jnp.concatenate(
                [xt[j * 256 : j * 256 + 128], xt[j * 256 + 128 : j * 256 + 256]],
                axis=1,
            )
            for j in range(KB // 256)
        ]
        o_ref[...] = jnp.concatenate(pieces, axis=0)

    return pl.pallas_call(
        tr,
        grid=(grid,),
        in_specs=[pl.BlockSpec((64, KB), lambda i: (0, i))],
        out_specs=pl.BlockSpec((KB // 2, _DP), lambda i: (i, 0)),
        out_shape=jax.ShapeDtypeStruct((n_lines, _DP), jnp.float32),
    )(wte_t)


def kernel(input_ids, wte):
    B, L = input_ids.shape
    V, D = wte.shape
    N = B * L
    idx_flat = input_ids.reshape(N).astype(jnp.int32)
    lines = _make_lines(wte.T, V)

    n_per_w = N // _NW
    n_blk = n_per_w // _CB
    assert n_per_w % _CB == 0 and n_blk % 2 == 0

    line_idx = ((idx_flat >> 8) << 7) | (idx_flat & 127)

    mesh = plsc.VectorSubcoreMesh(core_axis_name="c", subcore_axis_name="s")

    @functools.partial(
        pl.kernel,
        mesh=mesh,
        out_type=jax.ShapeDtypeStruct((N, _DP), jnp.float32),
        scratch_types=[
            pltpu.VMEM((n_per_w,), jnp.int32),
            pltpu.VMEM((_CB, _DP), jnp.float32),
            pltpu.VMEM((_CB, _DP), jnp.float32),
            pltpu.SemaphoreType.DMA,
            pltpu.SemaphoreType.DMA,
            pltpu.SemaphoreType.DMA,
            pltpu.SemaphoreType.DMA,
        ],
    )
    def emb(idx_hbm, table_hbm, out_hbm, idx_v, buf0, buf1, sg0, sg1, sw0, sw1):
        wid = lax.axis_index("s") * _NC + lax.axis_index("c")
        base = wid * n_per_w
        pltpu.sync_copy(idx_hbm.at[pl.ds(base, n_per_w)], idx_v)

        bufs = (buf0, buf1)
        sgs = (sg0, sg1)
        sws = (sw0, sw1)

        def fire_gathers(blk, buf, sem):
            off = blk * _CB
            for j in range(_G):
                pltpu.async_copy(
                    table_hbm.at[idx_v.at[pl.ds(off + j * _CH, _CH)]],
                    buf.at[pl.ds(j * _CH, _CH)],
                    sem,
                )

        def drain_gathers(buf, sem):
            pltpu.make_async_copy(table_hbm.at[pl.ds(0, _CB)], buf, sem).wait()

        def fire_writeout(blk, buf, sem):
            pltpu.async_copy(buf, out_hbm.at[pl.ds(base + blk * _CB, _CB)], sem)

        def drain_writeout(buf, sem):
            pltpu.make_async_copy(buf, out_hbm.at[pl.ds(base, _CB)], sem).wait()

        fire_gathers(0, buf0, sg0)

        def body(t, carry):
            for b in range(2):
                blk = 2 * t + b
                p, q = b % 2, (b + 1) % 2
                drain_gathers(bufs[p], sgs[p])

                @pl.when(jnp.logical_and(blk >= 1, blk + 1 < n_blk))
                def _():
                    drain_writeout(bufs[q], sws[q])

                @pl.when(blk + 1 < n_blk)
                def _():
                    fire_gathers(blk + 1, bufs[q], sgs[q])

                fire_writeout(blk, bufs[p], sws[p])
            return carry

        lax.fori_loop(0, n_blk // 2, body, 0)
        drain_writeout(bufs[0], sws[0])
        drain_writeout(bufs[1], sws[1])

    out = emb(line_idx, lines)
    upper = ((idx_flat >> 7) & 1)[:, None] == 1
    out64 = jnp.where(upper, out[:, D:], out[:, :D])
    return out64.reshape(B, L, D)


# speedup vs baseline: 1.2834x; 1.1280x over previous
import functools

import jax
import jax.numpy as jnp
from jax import lax
from jax.experimental import pallas as pl
from jax.experimental.pallas import tpu as pltpu
from jax.experimental.pallas import tpu_sc as plsc

_NC = 2
_NS = 16
_NW = _NC * _NS

_CH = 128
_G = 2
_CB = _CH * _G
_DP = 128


def _make_lines(wte_t, V):
    KB = 4096
    grid = (V + KB - 1) // KB
    n_lines = grid * (KB // 2)

    def tr(x_ref, o_ref):
        x = x_ref[...]
        eye = jnp.eye(64, dtype=x.dtype)
        xt = jax.lax.dot_general(
            x, eye, (((0,), (0,)), ((), ())),
            preferred_element_type=jnp.float32,
        )
        pieces = [
            jnp.concatenate(
                [xt[j * 256 : j * 256 + 128], xt[j * 256 + 128 : j * 256 + 256]],
                axis=1,
            )
            for j in range(KB // 256)
        ]
        o_ref[...] = jnp.concatenate(pieces, axis=0)

    return pl.pallas_call(
        tr,
        grid=(grid,),
        in_specs=[pl.BlockSpec((64, KB), lambda i: (0, i))],
        out_specs=pl.BlockSpec((KB // 2, _DP), lambda i: (i, 0)),
        out_shape=jax.ShapeDtypeStruct((n_lines, _DP), jnp.float32),
    )(wte_t)


def kernel(input_ids, wte):
    B, L = input_ids.shape
    V, D = wte.shape
    N = B * L
    idx_flat = input_ids.reshape(N).astype(jnp.int32)
    lines = _make_lines(wte.T, V)

    n_per_w = N // _NW
    n_blk = n_per_w // _CB
    assert n_per_w % _CB == 0 and n_blk % 2 == 0

    line_idx = ((idx_flat >> 8) << 7) | (idx_flat & 127)

    mesh = plsc.VectorSubcoreMesh(core_axis_name="c", subcore_axis_name="s")

    @functools.partial(
        pl.kernel,
        mesh=mesh,
        out_type=jax.ShapeDtypeStruct((N, _DP), jnp.float32),
        scratch_types=[
            pltpu.VMEM((n_per_w,), jnp.int32),
            pltpu.VMEM((_CB, _DP), jnp.float32),
            pltpu.VMEM((_CB, _DP), jnp.float32),
            pltpu.SemaphoreType.DMA,
            pltpu.SemaphoreType.DMA,
            pltpu.SemaphoreType.DMA,
            pltpu.SemaphoreType.DMA,
        ],
    )
    def emb(idx_hbm, table_hbm, out_hbm, idx_v, buf0, buf1, sg0, sg1, sw0, sw1):
        wid = lax.axis_index("s") * _NC + lax.axis_index("c")
        base = wid * n_per_w
        pltpu.sync_copy(idx_hbm.at[pl.ds(base, n_per_w)], idx_v)

        bufs = (buf0, buf1)
        sgs = (sg0, sg1)
        sws = (sw0, sw1)

        def fire_gathers(blk, buf, sem):
            off = blk * _CB
            for j in range(_G):
                pltpu.async_copy(
                    table_hbm.at[idx_v.at[pl.ds(off + j * _CH, _CH)]],
                    buf.at[pl.ds(j * _CH, _CH)],
                    sem,
                )

        def drain_gathers(buf, sem):
            pltpu.make_async_copy(table_hbm.at[pl.ds(0, _CB)], buf, sem).wait()

        def fire_writeout(blk, buf, sem):
            pltpu.async_copy(buf, out_hbm.at[pl.ds(base + blk * _CB, _CB)], sem)

        def drain_writeout(buf, sem):
            pltpu.make_async_copy(buf, out_hbm.at[pl.ds(base, _CB)], sem).wait()

        fire_gathers(0, buf0, sg0)

        def body(t, carry):
            for b in range(2):
                blk = 2 * t + b
                p, q = b % 2, (b + 1) % 2
                drain_gathers(bufs[p], sgs[p])

                @pl.when(jnp.logical_and(blk >= 1, blk + 1 < n_blk))
                def _():
                    drain_writeout(bufs[q], sws[q])

                @pl.when(blk + 1 < n_blk)
                def _():
                    fire_gathers(blk + 1, bufs[q], sgs[q])

                fire_writeout(blk, bufs[p], sws[p])
            return carry

        lax.fori_loop(0, n_blk // 2, body, 0)
        drain_writeout(bufs[0], sws[0])
        drain_writeout(bufs[1], sws[1])

    out = emb(line_idx, lines)
    upper = ((idx_flat >> 7) & 1)[:, None] == 1
    out64 = jnp.where(upper, out[:, D:], out[:, :D])
    return out64.reshape(B, L, D)


# speedup vs baseline: 2.0453x; 1.5936x over previous
import functools

import jax
import jax.numpy as jnp
from jax import lax
from jax.experimental import pallas as pl
from jax.experimental.pallas import tpu as pltpu
from jax.experimental.pallas import tpu_sc as plsc

_NC = 2
_NS = 16
_NW = _NC * _NS

_CH = 128
_G = 4
_CB = _CH * _G
_KB = 4096


def _make_rows(wte_t, V, D):
    grid = (V + _KB - 1) // _KB

    def tr(x_ref, o_ref):
        x = x_ref[...]
        eye = jnp.eye(D, dtype=x.dtype)
        xt = jax.lax.dot_general(
            x, eye, (((0,), (0,)), ((), ())),
            preferred_element_type=jnp.float32,
        )
        h = _KB // 2
        o_ref[...] = jnp.concatenate([xt[:h], xt[h:]], axis=1).reshape(_KB * D)

    return pl.pallas_call(
        tr,
        grid=(grid,),
        in_specs=[pl.BlockSpec((D, _KB), lambda i: (0, i))],
        out_specs=pl.BlockSpec((_KB * D,), lambda i: (i,)),
        out_shape=jax.ShapeDtypeStruct((grid * _KB * D,), jnp.float32),
    )(wte_t)


def kernel(input_ids, wte):
    B, L = input_ids.shape
    V, D = wte.shape
    N = B * L
    idx_flat = input_ids.T.reshape(N).astype(jnp.int32)
    rows_flat = _make_rows(wte.T, V, D)
    rows = rows_flat.reshape(rows_flat.shape[0] // D, D)

    ridx = (
        ((idx_flat >> 12) << 12)
        | ((idx_flat & 2047) << 1)
        | ((idx_flat >> 11) & 1)
    )

    b_per_w = B // _NW
    n_per_w = L * b_per_w
    n_blk = n_per_w // _CB
    assert b_per_w % _CB == 0 and n_blk % 2 == 0

    mesh = plsc.VectorSubcoreMesh(core_axis_name="c", subcore_axis_name="s")

    @functools.partial(
        pl.kernel,
        mesh=mesh,
        out_type=jax.ShapeDtypeStruct((L, B, D), jnp.float32),
        compiler_params=pltpu.CompilerParams(use_tc_tiling_on_sc=False),
        scratch_types=[
            pltpu.VMEM((n_per_w,), jnp.int32),
            pltpu.VMEM((_CB, D), jnp.float32),
            pltpu.VMEM((_CB, D), jnp.float32),
            pltpu.SemaphoreType.DMA,
            pltpu.SemaphoreType.DMA,
            pltpu.SemaphoreType.DMA,
            pltpu.SemaphoreType.DMA,
        ],
    )
    def emb(idx_hbm, table_hbm, out_hbm, idx_v, buf0, buf1, sg0, sg1, sw0, sw1):
        wid = lax.axis_index("s") * _NC + lax.axis_index("c")
        base_b = wid * b_per_w

        for l in range(L):
            pltpu.async_copy(
                idx_hbm.at[pl.ds(l * B + base_b, b_per_w)],
                idx_v.at[pl.ds(l * b_per_w, b_per_w)],
                sg0,
            )
        pltpu.make_async_copy(idx_hbm.at[pl.ds(0, n_per_w)], idx_v, sg0).wait()

        bufs = (buf0, buf1)
        sgs = (sg0, sg1)
        sws = (sw0, sw1)

        def fire_gathers(blk, buf, sem):
            off = blk * _CB
            for j in range(_G):
                pltpu.async_copy(
                    table_hbm.at[idx_v.at[pl.ds(off + j * _CH, _CH)]],
                    buf.at[pl.ds(j * _CH, _CH)],
                    sem,
                )

        def drain_gathers(buf, sem):
            pltpu.make_async_copy(table_hbm.at[pl.ds(0, _CB)], buf, sem).wait()

        def fire_writeout(blk, buf, sem):
            pltpu.async_copy(buf, out_hbm.at[blk, pl.ds(base_b, _CB)], sem)

        def drain_writeout(buf, sem):
            pltpu.make_async_copy(buf, out_hbm.at[0, pl.ds(base_b, _CB)], sem).wait()

        fire_gathers(0, buf0, sg0)

        def body(t, carry):
            for b in range(2):
                blk = 2 * t + b
                p, q = b % 2, (b + 1) % 2
                drain_gathers(bufs[p], sgs[p])

                @pl.when(jnp.logical_and(blk >= 1, blk + 1 < n_blk))
                def _():
                    drain_writeout(bufs[q], sws[q])

                @pl.when(blk + 1 < n_blk)
                def _():
                    fire_gathers(blk + 1, bufs[q], sgs[q])

                fire_writeout(blk, bufs[p], sws[p])
            return carry

        lax.fori_loop(0, n_blk // 2, body, 0)
        drain_writeout(bufs[0], sws[0])
        drain_writeout(bufs[1], sws[1])

    out = emb(ridx, rows)
    return out.transpose(1, 0, 2)


# speedup vs baseline: 2.5485x; 1.2460x over previous
import functools

import jax
import jax.numpy as jnp
from jax import lax
from jax.experimental import pallas as pl
from jax.experimental.pallas import tpu as pltpu
from jax.experimental.pallas import tpu_sc as plsc

_NC = 2
_NS = 16
_NW = _NC * _NS

_CH = 128
_G = 4
_CB = _CH * _G
_KB = 4096


def _make_rows(wte_t, V, D):
    grid = (V + _KB - 1) // _KB

    def tr(x_ref, o_ref):
        xt = x_ref[...].T
        h = _KB // 2
        o_ref[...] = jnp.concatenate([xt[:h], xt[h:]], axis=1).reshape(_KB * D)

    return pl.pallas_call(
        tr,
        grid=(grid,),
        in_specs=[pl.BlockSpec((D, _KB), lambda i: (0, i))],
        out_specs=pl.BlockSpec((_KB * D,), lambda i: (i,)),
        out_shape=jax.ShapeDtypeStruct((grid * _KB * D,), jnp.float32),
    )(wte_t)


def kernel(input_ids, wte):
    B, L = input_ids.shape
    V, D = wte.shape
    N = B * L
    idx_flat = input_ids.T.reshape(N).astype(jnp.int32)
    rows_flat = _make_rows(wte.T, V, D)
    rows = rows_flat.reshape(rows_flat.shape[0] // D, D)

    ridx = (
        ((idx_flat >> 12) << 12)
        | ((idx_flat & 2047) << 1)
        | ((idx_flat >> 11) & 1)
    )

    b_per_w = B // _NW
    n_per_w = L * b_per_w
    n_blk = n_per_w // _CB
    assert b_per_w % _CB == 0 and n_blk % 2 == 0

    mesh = plsc.VectorSubcoreMesh(core_axis_name="c", subcore_axis_name="s")

    @functools.partial(
        pl.kernel,
        mesh=mesh,
        out_type=jax.ShapeDtypeStruct((L, B, D), jnp.float32),
        compiler_params=pltpu.CompilerParams(use_tc_tiling_on_sc=False),
        scratch_types=[
            pltpu.VMEM((n_per_w,), jnp.int32),
            pltpu.VMEM((_CB, D), jnp.float32),
            pltpu.VMEM((_CB, D), jnp.float32),
            pltpu.SemaphoreType.DMA,
            pltpu.SemaphoreType.DMA,
            pltpu.SemaphoreType.DMA,
            pltpu.SemaphoreType.DMA,
        ],
    )
    def emb(idx_hbm, table_hbm, out_hbm, idx_v, buf0, buf1, sg0, sg1, sw0, sw1):
        wid = lax.axis_index("s") * _NC + lax.axis_index("c")
        base_b = wid * b_per_w

        for l in range(L):
            pltpu.async_copy(
                idx_hbm.at[pl.ds(l * B + base_b, b_per_w)],
                idx_v.at[pl.ds(l * b_per_w, b_per_w)],
                sg0,
            )
        pltpu.make_async_copy(idx_hbm.at[pl.ds(0, n_per_w)], idx_v, sg0).wait()

        bufs = (buf0, buf1)
        sgs = (sg0, sg1)
        sws = (sw0, sw1)

        def fire_gathers(blk, buf, sem):
            off = blk * _CB
            for j in range(_G):
                pltpu.async_copy(
                    table_hbm.at[idx_v.at[pl.ds(off + j * _CH, _CH)]],
                    buf.at[pl.ds(j * _CH, _CH)],
                    sem,
                )

        def drain_gathers(buf, sem):
            pltpu.make_async_copy(table_hbm.at[pl.ds(0, _CB)], buf, sem).wait()

        def fire_writeout(blk, buf, sem):
            pltpu.async_copy(buf, out_hbm.at[blk, pl.ds(base_b, _CB)], sem)

        def drain_writeout(buf, sem):
            pltpu.make_async_copy(buf, out_hbm.at[0, pl.ds(base_b, _CB)], sem).wait()

        fire_gathers(0, buf0, sg0)

        def body(t, carry):
            for b in range(2):
                blk = 2 * t + b
                p, q = b % 2, (b + 1) % 2
                drain_gathers(bufs[p], sgs[p])

                @pl.when(jnp.logical_and(blk >= 1, blk + 1 < n_blk))
                def _():
                    drain_writeout(bufs[q], sws[q])

                @pl.when(blk + 1 < n_blk)
                def _():
                    fire_gathers(blk + 1, bufs[q], sgs[q])

                fire_writeout(blk, bufs[p], sws[p])
            return carry

        lax.fori_loop(0, n_blk // 2, body, 0)
        drain_writeout(bufs[0], sws[0])
        drain_writeout(bufs[1], sws[1])

    out = emb(ridx, rows)
    return out.transpose(1, 0, 2)
